# Initial kernel scaffold; baseline (speedup 1.0000x reference)
import jax
import jax.numpy as jnp
from jax.experimental import pallas as pl


def kernel(x, g1_fc1_W, g1_fc1_b, g1_fc1_g, g1_fc1_be, g1_mr_W, g1_mr_b, g1_fc2_W, g1_fc2_b, g1_fc2_g, g1_fc2_be, g1_rel_pos, g2_fc1_W, g2_fc1_b, g2_fc1_g, g2_fc1_be, g2_mr_W, g2_mr_b, g2_fc2_W, g2_fc2_b, g2_fc2_g, g2_fc2_be, g2_rel_pos, f1_fc1_W, f1_fc1_b, f1_fc2_W, f1_fc2_b, f2_fc1_W, f2_fc1_b, f2_fc2_W, f2_fc2_b):
    raise NotImplementedError("write your pallas kernel here")


# speedup vs baseline: 363.8653x; 363.8653x over previous
import functools

import jax
import jax.numpy as jnp
from jax.experimental import pallas as pl
from jax.experimental.pallas import tpu as pltpu
from jax.experimental.pallas import tpu_sc as plsc

B = 2
C = 96
N = 3136
NT = B * N
K = 9
TN = 448
TM = 392
NUM_IDX = K * NT
NUM_IDX_PAD = 57344
SC_WINDOW = 128

_F32 = jnp.float32


def _bdot(a, b):
    return jax.lax.dot_general(a.astype(jnp.bfloat16), b.astype(jnp.bfloat16),
                               (((1,), (0,)), ((), ())),
                               preferred_element_type=_F32)


def _b3dot(a, b):
    ah = a.astype(jnp.bfloat16)
    al = (a - ah.astype(_F32)).astype(jnp.bfloat16)
    bh = b.astype(jnp.bfloat16)
    bl = (b - bh.astype(_F32)).astype(jnp.bfloat16)

    def d(u, v):
        return jax.lax.dot_general(u, v, (((1,), (0,)), ((), ())),
                                   preferred_element_type=_F32)

    return d(ah, bl) + d(al, bh) + d(ah, bh)


def _gelu(x):
    return 0.5 * x * (1.0 + jax.lax.erf(x * 0.7071067811865476))


def _inorm2d(x, cc, eps=1e-5):
    x3 = x.reshape(B, N, cc)
    m = jnp.mean(x3, axis=1, keepdims=True)
    v = jnp.mean((x3 - m) ** 2, axis=1, keepdims=True)
    return ((x3 - m) / jnp.sqrt(v + eps)).reshape(NT, cc)


def _bnorm2d(x, g, b, eps=1e-5):
    m = jnp.mean(x, axis=0, keepdims=True)
    v = jnp.mean((x - m) ** 2, axis=0, keepdims=True)
    return (x - m) / jnp.sqrt(v + eps) * g + b


def _prep_body(x_ref, w_ref, b_ref, g_ref, be_ref,
               y_ref, ypad_ref, xn_ref, xnt_ref, sqt_ref, sql_ref):
    x = x_ref[...]
    y = _bdot(x, w_ref[...]) + b_ref[...]
    y = _bnorm2d(y, g_ref[...], be_ref[...])
    y_ref[...] = y
    ypad_ref[...] = jnp.concatenate(
        [y, jnp.zeros((NT, 128 - C), _F32)], axis=1)
    nrm = jnp.sqrt(jnp.sum(y * y, axis=1, keepdims=True))
    xn = y / jnp.maximum(nrm, 1e-12)
    xn3 = xn.reshape(B, N, C)
    xn_ref[...] = xn3
    xnt = jnp.swapaxes(xn3, 1, 2)
    xnt_ref[...] = xnt
    sqt = jnp.sum(xn3 * xn3, axis=2, keepdims=True)
    sqt_ref[...] = sqt
    sql_ref[...] = jnp.swapaxes(sqt, 1, 2)


def _prep(x2d, w_t, bb, g, be):
    out_shape = [
        jax.ShapeDtypeStruct((NT, C), _F32),
        jax.ShapeDtypeStruct((NT, 128), _F32),
        jax.ShapeDtypeStruct((B, N, C), _F32),
        jax.ShapeDtypeStruct((B, C, N), _F32),
        jax.ShapeDtypeStruct((B, N, 1), _F32),
        jax.ShapeDtypeStruct((B, 1, N), _F32),
    ]
    return pl.pallas_call(_prep_body, out_shape=out_shape)(
        x2d, w_t, bb.reshape(1, C), g.reshape(1, C), be.reshape(1, C))


def _topk_body(xn_ref, xnt_ref, sqt_ref, sql_ref, rel_ref, idx_ref):
    b = pl.program_id(1)
    rows = xn_ref[0]
    xnt = xnt_ref[0]
    xx = _bdot(rows, xnt)
    inner = -2.0 * xx
    dist = ((sqt_ref[0] + inner) + sql_ref[0]) + rel_ref[0]
    col = jax.lax.broadcasted_iota(jnp.int32, (TN, N), 1)
    vals = dist
    for j in range(K):
        m = jnp.min(vals, axis=1, keepdims=True)
        sel = jnp.where(vals == m, col, N)
        idx = jnp.min(sel, axis=1, keepdims=True)
        idx_ref[0, :, pl.ds(j, 1)] = idx + b * N
        vals = jnp.where(col == idx, jnp.inf, vals)


def _topk(xn, xnt, sqt, sql, rel_pos):
    ntiles = N // TN
    grid = (ntiles, B)
    return pl.pallas_call(
        _topk_body,
        grid=grid,
        in_specs=[
            pl.BlockSpec((1, TN, C), lambda t, b: (b, t, 0)),
            pl.BlockSpec((1, C, N), lambda t, b: (b, 0, 0)),
            pl.BlockSpec((1, TN, 1), lambda t, b: (b, t, 0)),
            pl.BlockSpec((1, 1, N), lambda t, b: (b, 0, 0)),
            pl.BlockSpec((1, TN, N), lambda t, b: (0, t, 0)),
        ],
        out_specs=pl.BlockSpec((1, TN, 16), lambda t, b: (b, t, 0)),
        out_shape=jax.ShapeDtypeStruct((B, N, 16), jnp.int32),
    )(xn, xnt, sqt, sql, rel_pos)


def _sc_gather(yflat, idx):
    mesh = plsc.VectorSubcoreMesh(core_axis_name="c", subcore_axis_name="s")

    @functools.partial(
        pl.kernel,
        out_type=jax.ShapeDtypeStruct((NUM_IDX_PAD, 128), _F32),
        mesh=mesh)
    def k(x_hbm, i_hbm, o_hbm):
        def body(i_vmem, o_vmem):
            pltpu.sync_copy(x_hbm.at[i_vmem.at[0]], o_vmem)

        pltpu.emit_pipeline(
            body,
            grid=(NUM_IDX_PAD // SC_WINDOW,),
            in_specs=[pl.BlockSpec((1, SC_WINDOW), lambda i: (0, i))],
            out_specs=[pl.BlockSpec((SC_WINDOW, 128), lambda i: (i, 0))],
            core_axis_name=("c", "s"),
            dimension_semantics=(pltpu.PARALLEL,),
        )(i_hbm, o_hbm)

    return k(yflat, idx)


def _mr_a_body(y_ref, bdi_ref, b_ref, *refs):
    g_refs = refs[:K]
    y2_ref = refs[K]
    y0 = y_ref[0]
    mx = g_refs[0][:, :C]
    for k in range(1, K):
        mx = jnp.maximum(mx, g_refs[k][:, :C])
    xj = mx - y0
    xc = jnp.stack([y0, xj], axis=-1).reshape(TM, 2 * C)
    y2 = _bdot(xc, bdi_ref[...]) + b_ref[...]
    y2_ref[0] = y2


def _mr_a(y3, bdi, bvec, gathered):
    tiles = N // TM
    grid = (B, tiles)
    gather_specs = [
        pl.BlockSpec((TM, 128),
                     functools.partial(lambda k, b, t: (k * (NT // TM) + b * tiles + t, 0), k))
        for k in range(K)
    ]
    return pl.pallas_call(
        _mr_a_body,
        grid=grid,
        in_specs=[
            pl.BlockSpec((1, TM, C), lambda b, t: (b, t, 0)),
            pl.BlockSpec((2 * C, 2 * C), lambda b, t: (0, 0)),
            pl.BlockSpec((1, 2 * C), lambda b, t: (0, 0)),
        ] + gather_specs,
        out_specs=pl.BlockSpec((1, TM, 2 * C), lambda b, t: (b, t, 0)),
        out_shape=jax.ShapeDtypeStruct((B, N, 2 * C), _F32),
    )(y3, bdi, bvec, *([gathered] * K))


def _mr_b_body(y2_ref, w2_ref, b2_ref, g2_ref, be2_ref, sc_ref, o_ref):
    y2 = y2_ref[...].reshape(NT, 2 * C)
    h = _gelu(_inorm2d(y2, 2 * C))
    y3 = _bdot(h, w2_ref[...]) + b2_ref[...]
    y3 = _bnorm2d(y3, g2_ref[...], be2_ref[...])
    o_ref[...] = y3 + sc_ref[...]


def _mr_b(y2, w2t, b2, g2, be2, sc):
    return pl.pallas_call(
        _mr_b_body,
        out_shape=jax.ShapeDtypeStruct((NT, C), _F32),
    )(y2, w2t, b2.reshape(1, C), g2.reshape(1, C), be2.reshape(1, C), sc)


def _ffn_body(x_ref, w1_ref, b1_ref, w2_ref, b2_ref, x0_ref, o_ref, *, mode):
    x = x_ref[...]
    h = _bdot(x, w1_ref[...]) + b1_ref[...]
    h = _gelu(_inorm2d(h, 4 * C))
    y = _bdot(h, w2_ref[...]) + b2_ref[...]
    y = _inorm2d(y, C)
    y = y + x
    y = _inorm2d(y, C)
    if mode == "relu":
        o_ref[...] = jnp.maximum(y, 0.0)
    else:
        o_ref[...] = x0_ref[...] + y


def _ffn(x2d, w1t, b1, w2t, b2, x0, mode):
    body = functools.partial(_ffn_body, mode=mode)
    return pl.pallas_call(
        body,
        out_shape=jax.ShapeDtypeStruct((NT, C), _F32),
    )(x2d, w1t, b1.reshape(1, 4 * C), w2t, b2.reshape(1, C), x0)


def _grapher(x2d, fc1_W, fc1_b, fc1_g, fc1_be, mr_W, mr_b,
             fc2_W, fc2_b, fc2_g, fc2_be, rel_pos):
    y2d, ypad, xn, xnt, sqt, sql = _prep(x2d, fc1_W.T, fc1_b, fc1_g, fc1_be)
    idx16 = _topk(xn, xnt, sqt, sql, rel_pos)

    idx9 = idx16[:, :, :K]
    idx_flat = jnp.transpose(idx9, (2, 0, 1)).reshape(1, NUM_IDX)
    idx_flat = jnp.concatenate(
        [idx_flat, jnp.zeros((1, NUM_IDX_PAD - NUM_IDX), jnp.int32)], axis=1)
    gathered = _sc_gather(ypad, idx_flat)

    wg = mr_W.reshape(4, 2 * C // 4, 2 * C // 4)
    bdi = jnp.zeros((2 * C, 2 * C), _F32)
    for g in range(4):
        bdi = bdi.at[48 * g:48 * (g + 1), 48 * g:48 * (g + 1)].set(wg[g].T)

    y3 = y2d.reshape(B, N, C)
    y2 = _mr_a(y3, bdi, mr_b.reshape(1, 2 * C), gathered)
    return _mr_b(y2, fc2_W.T, fc2_b, fc2_g, fc2_be, x2d)


def kernel(x, g1_fc1_W, g1_fc1_b, g1_fc1_g, g1_fc1_be, g1_mr_W, g1_mr_b,
           g1_fc2_W, g1_fc2_b, g1_fc2_g, g1_fc2_be, g1_rel_pos,
           g2_fc1_W, g2_fc1_b, g2_fc1_g, g2_fc1_be, g2_mr_W, g2_mr_b,
           g2_fc2_W, g2_fc2_b, g2_fc2_g, g2_fc2_be, g2_rel_pos,
           f1_fc1_W, f1_fc1_b, f1_fc2_W, f1_fc2_b,
           f2_fc1_W, f2_fc1_b, f2_fc2_W, f2_fc2_b):
    H = 56
    xt = x.reshape(B, C, N).transpose(0, 2, 1).reshape(NT, C)

    z1 = _grapher(xt, g1_fc1_W, g1_fc1_b, g1_fc1_g, g1_fc1_be,
                  g1_mr_W, g1_mr_b, g1_fc2_W, g1_fc2_b, g1_fc2_g, g1_fc2_be,
                  g1_rel_pos)
    z2 = _ffn(z1, f1_fc1_W.T, f1_fc1_b, f1_fc2_W.T, f1_fc2_b, z1, "relu")
    z3 = _grapher(z2, g2_fc1_W, g2_fc1_b, g2_fc1_g, g2_fc1_be,
                  g2_mr_W, g2_mr_b, g2_fc2_W, g2_fc2_b, g2_fc2_g, g2_fc2_be,
                  g2_rel_pos)
    z4 = _ffn(z3, f2_fc1_W.T, f2_fc1_b, f2_fc2_W.T, f2_fc2_b, xt, "addx")

    return z4.reshape(B, N, C).transpose(0, 2, 1).reshape(B, C, H, H)
